# baseline (speedup 1.0000x reference)
import functools

import jax
import jax.numpy as jnp
from jax import lax
from jax.experimental import pallas as pl
from jax.experimental.pallas import tpu as pltpu
from jax.experimental.pallas import tpu_sc as plsc

D_MODEL = 64
NUM_CORES = 2
NUM_SUBCORES = 16
NW = NUM_CORES * NUM_SUBCORES
LANE = 128
NBUF = 5


@functools.partial(jax.jit, static_argnames=("bsz", "seq"))
def _emb_lookup(table, xt, bsz, seq):
    mesh = plsc.VectorSubcoreMesh(
        core_axis_name="c", subcore_axis_name="s",
        num_cores=NUM_CORES, num_subcores=NUM_SUBCORES)
    d8 = D_MODEL // 8

    @functools.partial(
        pl.kernel,
        out_type=jax.ShapeDtypeStruct(
            (seq, d8, bsz // LANE, 8, LANE), jnp.float32),
        mesh=mesh,
        scratch_types=[
            pltpu.VMEM((seq, LANE), jnp.int32),
            pltpu.VMEM((NBUF, LANE, D_MODEL), jnp.float32),
            pltpu.VMEM((NBUF, d8, 8, LANE), jnp.float32),
            pltpu.SemaphoreType.DMA,
            pltpu.SemaphoreType.DMA((NBUF,)),
            pltpu.SemaphoreType.DMA((NBUF,)),
        ],
        compiler_params=pltpu.CompilerParams(
            use_tc_tiling_on_sc=False, needs_layout_passes=False),
    )
    def k(table_hbm, xt_hbm, out_hbm, idx_t, rows_v, trans_v,
          isem, gsems, osems):
        wid = lax.axis_index("s") * NUM_CORES + lax.axis_index("c")
        base = wid * LANE

        cp = pltpu.make_async_copy(
            xt_hbm.at[:, pl.ds(base, LANE)], idx_t, isem)
        cp.start()
        cp.wait()

        def g_copy(t, s):
            return pltpu.make_async_copy(
                table_hbm.at[idx_t.at[t]], rows_v.at[s], gsems.at[s])

        def o_copy(t, s):
            return pltpu.make_async_copy(
                trans_v.at[s], out_hbm.at[t, :, wid], osems.at[s])

        iota = lax.broadcasted_iota(jnp.int32, (16,), 0)
        rowsel = [iota + c * 16 for c in range(LANE // 16)]

        for s in range(NBUF):
            g_copy(s, s).start()

        n_rounds = seq // NBUF

        def round_body(r, carry):
            for s in range(NBUF):
                t = r * NBUF + s
                g_copy(t, s).wait()

                def j_body(j, c2):
                    for dr in range(8):
                        dsel = jnp.full((16,), j * 8 + dr, jnp.int32)
                        for c in range(LANE // 16):
                            v = plsc.load_gather(
                                rows_v.at[s], [rowsel[c], dsel])
                            trans_v[s, j, dr, pl.ds(c * 16, 16)] = v
                    return c2

                lax.fori_loop(0, d8, j_body, 0)
                o_copy(t, s).start()

            for s in range(NBUF):
                t = r * NBUF + s
                o_copy(t, s).wait()
                tn = t + NBUF

                @pl.when(tn < seq)
                def _():
                    g_copy(tn, s).start()

            return carry

        lax.fori_loop(0, n_rounds, round_body, 0)

    return k(table, xt)


def kernel(x, table):
    bsz, seq = x.shape
    out5 = _emb_lookup(table, x.T, bsz, seq)
    return out5.transpose(2, 4, 0, 1, 3).reshape(bsz, seq, D_MODEL)


# speedup vs baseline: 6.4921x; 6.4921x over previous
import functools

import jax
import jax.numpy as jnp
from jax import lax
from jax.experimental import pallas as pl
from jax.experimental.pallas import tpu as pltpu
from jax.experimental.pallas import tpu_sc as plsc

D_MODEL = 64
NUM_CORES = 2
NUM_SUBCORES = 16
NW = NUM_CORES * NUM_SUBCORES
LANE = 128
NBUF = 5


@functools.partial(jax.jit, static_argnames=("bsz", "seq"))
def _emb_lookup(table, xt, bsz, seq):
    mesh = plsc.VectorSubcoreMesh(
        core_axis_name="c", subcore_axis_name="s",
        num_cores=NUM_CORES, num_subcores=NUM_SUBCORES)
    d8 = D_MODEL // 8

    @functools.partial(
        pl.kernel,
        out_type=jax.ShapeDtypeStruct(
            (seq, d8, bsz // LANE, 8, LANE), jnp.float32),
        mesh=mesh,
        scratch_types=[
            pltpu.VMEM((seq, LANE), jnp.int32),
            pltpu.VMEM((NBUF, LANE, D_MODEL), jnp.float32),
            pltpu.VMEM((NBUF, d8, 8, LANE), jnp.float32),
            pltpu.SemaphoreType.DMA,
            pltpu.SemaphoreType.DMA((NBUF,)),
            pltpu.SemaphoreType.DMA((NBUF,)),
        ],
        compiler_params=pltpu.CompilerParams(
            use_tc_tiling_on_sc=False, needs_layout_passes=False),
    )
    def k(table_hbm, xt_hbm, out_hbm, idx_t, rows_v, trans_v,
          isem, gsems, osems):
        wid = lax.axis_index("s") * NUM_CORES + lax.axis_index("c")
        base = wid * LANE

        cp = pltpu.make_async_copy(
            xt_hbm.at[:, pl.ds(base, LANE)], idx_t, isem)
        cp.start()
        cp.wait()

        def g_copy(t, s):
            return pltpu.make_async_copy(
                table_hbm.at[idx_t.at[t]], rows_v.at[s], gsems.at[s])

        def o_copy(t, s):
            return pltpu.make_async_copy(
                trans_v.at[s], out_hbm.at[t, :, wid], osems.at[s])

        iota = lax.broadcasted_iota(jnp.int32, (16,), 0)
        rowsel = [iota + c * 16 for c in range(LANE // 16)]

        for s in range(NBUF):
            g_copy(s, s).start()

        n_rounds = seq // NBUF

        def round_body(r, carry):
            for s in range(NBUF):
                t = r * NBUF + s
                g_copy(t, s).wait()

                @functools.partial(
                    plsc.parallel_loop, 0, D_MODEL, unroll=8)
                def _transpose(d):
                    dsel = jnp.full((16,), d, jnp.int32)
                    for c in range(LANE // 16):
                        v = plsc.load_gather(
                            rows_v.at[s], [rowsel[c], dsel])
                        trans_v[s, d // 8, d % 8, pl.ds(c * 16, 16)] = v

                o_copy(t, s).start()

            for s in range(NBUF):
                t = r * NBUF + s
                o_copy(t, s).wait()
                tn = t + NBUF

                @pl.when(tn < seq)
                def _():
                    g_copy(tn, s).start()

            return carry

        lax.fori_loop(0, n_rounds, round_body, 0)

    return k(table, xt)


def kernel(x, table):
    bsz, seq = x.shape
    out5 = _emb_lookup(table, x.T, bsz, seq)
    return out5.transpose(2, 4, 0, 1, 3).reshape(bsz, seq, D_MODEL)
